# baseline (speedup 1.0000x reference)
import functools

import jax
import jax.numpy as jnp
from jax import lax
from jax.experimental import pallas as pl
from jax.experimental.pallas import tpu as pltpu
from jax.experimental.pallas import tpu_sc as plsc

_PARTS = 24
_NB = 3
_LAG = 1


def kernel(features):
    F, B, C, H, W = features.shape
    R = C * H
    rows = R // _PARTS
    x4 = jnp.reshape(features, (F, B, R, W))

    info = plsc.get_sparse_core_info()
    nw = info.num_cores * info.num_subcores
    n_chunks = F * B * _PARTS
    per_w = n_chunks // nw

    mesh = plsc.VectorSubcoreMesh(core_axis_name="c", subcore_axis_name="s")

    @functools.partial(
        pl.kernel,
        out_type=jax.ShapeDtypeStruct((B, F, R, W), features.dtype),
        mesh=mesh,
        scratch_types=[
            pltpu.VMEM((_NB, rows, W), features.dtype),
            pltpu.SemaphoreType.DMA((_NB,)),
            pltpu.SemaphoreType.DMA((_NB,)),
        ],
    )
    def sc_copy(x_hbm, o_hbm, buf, in_sem, out_sem):
        wid = lax.axis_index("s") * info.num_cores + lax.axis_index("c")

        def refs(t, s):
            g = wid * per_w + t
            i = g // (B * _PARTS)
            j = (g // _PARTS) % B
            p = g % _PARTS
            src = x_hbm.at[i, j, pl.ds(p * rows, rows)]
            dst = o_hbm.at[j, i, pl.ds(p * rows, rows)]
            return (
                pltpu.make_async_copy(src, buf.at[s], in_sem.at[s]),
                pltpu.make_async_copy(buf.at[s], dst, out_sem.at[s]),
            )

        for t in range(min(_NB, per_w)):
            refs(t, t % _NB)[0].start()
        waited = [False] * per_w
        for t in range(per_w):
            s = t % _NB
            cin, cout = refs(t, s)
            cin.wait()
            cout.start()
            m = t + _NB - _LAG
            if _NB <= m < per_w:
                refs(m - _NB, m % _NB)[1].wait()
                waited[m - _NB] = True
                refs(m, m % _NB)[0].start()
        for t in range(per_w):
            if not waited[t]:
                refs(t, t % _NB)[1].wait()

    out = sc_copy(x4)
    return jnp.reshape(out, (B, F * C * H, W))


# speedup vs baseline: 1.0129x; 1.0129x over previous
import functools

import jax
import jax.numpy as jnp
from jax import lax
from jax.experimental import pallas as pl
from jax.experimental.pallas import tpu as pltpu
from jax.experimental.pallas import tpu_sc as plsc

_PARTS = 16
_NB = 2
_LAG = 1


def kernel(features):
    F, B, C, H, W = features.shape
    R = C * H
    rows = R // _PARTS
    x4 = jnp.reshape(features, (F, B, R, W))

    info = plsc.get_sparse_core_info()
    nw = info.num_cores * info.num_subcores
    n_chunks = F * B * _PARTS
    per_w = n_chunks // nw

    mesh = plsc.VectorSubcoreMesh(core_axis_name="c", subcore_axis_name="s")

    @functools.partial(
        pl.kernel,
        out_type=jax.ShapeDtypeStruct((B, F, R, W), features.dtype),
        mesh=mesh,
        scratch_types=[
            pltpu.VMEM((_NB, rows, W), features.dtype),
            pltpu.SemaphoreType.DMA((_NB,)),
            pltpu.SemaphoreType.DMA((_NB,)),
        ],
    )
    def sc_copy(x_hbm, o_hbm, buf, in_sem, out_sem):
        wid = lax.axis_index("s") * info.num_cores + lax.axis_index("c")

        def refs(t, s):
            g = wid * per_w + t
            i = g // (B * _PARTS)
            j = (g // _PARTS) % B
            p = g % _PARTS
            src = x_hbm.at[i, j, pl.ds(p * rows, rows)]
            dst = o_hbm.at[j, i, pl.ds(p * rows, rows)]
            return (
                pltpu.make_async_copy(src, buf.at[s], in_sem.at[s]),
                pltpu.make_async_copy(buf.at[s], dst, out_sem.at[s]),
            )

        for t in range(min(_NB, per_w)):
            refs(t, t % _NB)[0].start()
        waited = [False] * per_w
        for t in range(per_w):
            s = t % _NB
            cin, cout = refs(t, s)
            cin.wait()
            cout.start()
            m = t + _NB - _LAG
            if _NB <= m < per_w:
                refs(m - _NB, m % _NB)[1].wait()
                waited[m - _NB] = True
                refs(m, m % _NB)[0].start()
        for t in range(per_w):
            if not waited[t]:
                refs(t, t % _NB)[1].wait()

    out = sc_copy(x4)
    return jnp.reshape(out, (B, F * C * H, W))


# speedup vs baseline: 1.0187x; 1.0057x over previous
import functools

import jax
import jax.numpy as jnp
from jax import lax
from jax.experimental import pallas as pl
from jax.experimental.pallas import tpu as pltpu
from jax.experimental.pallas import tpu_sc as plsc

_PARTS = 16
_NB = 2
_LAG = 1


def kernel(features):
    F, B, C, H, W = features.shape
    R = C * H
    rows = R // _PARTS
    x4 = jnp.reshape(features, (F, B, R, W))

    info = plsc.get_sparse_core_info()
    nw = info.num_cores * info.num_subcores
    n_chunks = F * B * _PARTS
    per_w = n_chunks // nw

    mesh = plsc.VectorSubcoreMesh(core_axis_name="c", subcore_axis_name="s")

    @functools.partial(
        pl.kernel,
        out_type=jax.ShapeDtypeStruct((B, F, R, W), features.dtype),
        mesh=mesh,
        scratch_types=[
            pltpu.VMEM((_NB, rows, W), features.dtype),
            pltpu.SemaphoreType.DMA((_NB,)),
            pltpu.SemaphoreType.DMA((_NB,)),
        ],
    )
    def sc_copy(x_hbm, o_hbm, buf, in_sem, out_sem):
        wid = lax.axis_index("s") * info.num_cores + lax.axis_index("c")

        def refs(t, s):
            g = wid * per_w + t
            j = g // (F * _PARTS)
            i = (g // _PARTS) % F
            p = g % _PARTS
            src = x_hbm.at[i, j, pl.ds(p * rows, rows)]
            dst = o_hbm.at[j, i, pl.ds(p * rows, rows)]
            return (
                pltpu.make_async_copy(src, buf.at[s], in_sem.at[s]),
                pltpu.make_async_copy(buf.at[s], dst, out_sem.at[s]),
            )

        for t in range(min(_NB, per_w)):
            refs(t, t % _NB)[0].start()
        waited = [False] * per_w
        for t in range(per_w):
            s = t % _NB
            cin, cout = refs(t, s)
            cin.wait()
            cout.start()
            m = t + _NB - _LAG
            if _NB <= m < per_w:
                refs(m - _NB, m % _NB)[1].wait()
                waited[m - _NB] = True
                refs(m, m % _NB)[0].start()
        for t in range(per_w):
            if not waited[t]:
                refs(t, t % _NB)[1].wait()

    out = sc_copy(x4)
    return jnp.reshape(out, (B, F * C * H, W))
